# baseline (speedup 1.0000x reference)
import functools

import jax
import jax.numpy as jnp
from jax import lax
from jax.experimental import pallas as pl
from jax.experimental.pallas import tpu as pltpu

NUM_SEG = 33
F = 128
N = 100000
BR = 5000
NB = N // BR


def _pool_body(depths_ref, emb_ref, out_ref, sum_s, max_s, cnt_s, *, num_blocks):
    i = pl.program_id(0)

    @pl.when(i == 0)
    def _init():
        sum_s[...] = jnp.zeros_like(sum_s)
        cnt_s[...] = jnp.zeros_like(cnt_s)
        max_s[...] = jnp.full_like(max_s, -jnp.inf)

    d2 = depths_ref[0, :, :]
    emb = emb_ref[...]

    dbc = jnp.broadcast_to(d2, (BR, F))

    seg_ids = lax.broadcasted_iota(jnp.int32, (1, NUM_SEG), 1)
    oh = (dbc[:, :NUM_SEG] == seg_ids).astype(jnp.float32)

    dims = (((0,), (0,)), ((), ()))
    sum_s[...] += lax.dot_general(oh, emb, dims,
                                  preferred_element_type=jnp.float32)
    cnt_s[...] += jnp.sum(oh, axis=0)[:, None]
    dmin = jnp.min(d2)
    dmax = jnp.max(d2)
    rows = lax.broadcasted_iota(jnp.int32, (NUM_SEG, 1), 0)
    ninf = jnp.float32(-jnp.inf)

    def _smax(s, c):
        masked = jnp.where(dbc == s, emb, ninf)
        blk = jnp.max(masked, axis=0)
        sel = rows == s
        max_s[...] = jnp.where(sel, jnp.maximum(max_s[...], blk[None, :]),
                               max_s[...])
        return c

    lax.fori_loop(dmin, dmax + 1, _smax, 0)

    @pl.when(i == num_blocks - 1)
    def _finish():
        cnt = cnt_s[...]
        mean = sum_s[...] / jnp.maximum(cnt, 1.0)
        nonempty = cnt > 0.0
        out_ref[:, :F] = jnp.where(nonempty, mean, 0.0)
        out_ref[:, F:] = jnp.where(nonempty, max_s[...], 0.0)


def kernel(node_embeddings, node_depths, max_depth):
    depths3 = jnp.minimum(node_depths, max_depth).astype(jnp.int32).reshape(NB, BR, 1)
    out = pl.pallas_call(
        functools.partial(_pool_body, num_blocks=NB),
        grid=(NB,),
        in_specs=[
            pl.BlockSpec((1, BR, 1), lambda i: (i, 0, 0)),
            pl.BlockSpec((BR, F), lambda i: (i, 0)),
        ],
        out_specs=pl.BlockSpec((NUM_SEG, 2 * F), lambda i: (0, 0)),
        out_shape=jax.ShapeDtypeStruct((NUM_SEG, 2 * F), jnp.float32),
        scratch_shapes=[
            pltpu.VMEM((NUM_SEG, F), jnp.float32),
            pltpu.VMEM((NUM_SEG, F), jnp.float32),
            pltpu.VMEM((NUM_SEG, 1), jnp.float32),
        ],
    )(depths3, node_embeddings)
    return out
